# Initial kernel scaffold; baseline (speedup 1.0000x reference)
import jax
import jax.numpy as jnp
from jax.experimental import pallas as pl


def kernel(loss, prev_loss, weights, param_0, param_1, param_2, param_3, param_4, param_5, param_6, param_7, param_8, param_9, param_10, param_11, param_12, param_13, param_14, param_15):
    raise NotImplementedError("write your pallas kernel here")


# speedup vs baseline: 1.3047x; 1.3047x over previous
import jax
import jax.numpy as jnp
from jax.experimental import pallas as pl
from jax.experimental.pallas import tpu as pltpu

_ROWS = 2048
_COLS = 2048
_BLK = 128


def _tc_body(w_ref, p0, p1, p2, p3, p4, p5, p6, p7, p8, p9, p10, p11,
             p12, p13, p14, p15, o18, o19, o20, o21, o22, o23):
    a = p0[...] * w_ref[2] + p4[...] * w_ref[6] + p8[...] * w_ref[10] + p12[...] * w_ref[14]
    b = p1[...] * w_ref[3] + p5[...] * w_ref[7] + p9[...] * w_ref[11] + p13[...] * w_ref[15]
    c = p2[...] * w_ref[4] + p6[...] * w_ref[8] + p10[...] * w_ref[12] + p14[...] * w_ref[16]
    d = p3[...] * w_ref[5] + p7[...] * w_ref[9] + p11[...] * w_ref[13] + p15[...] * w_ref[17]
    o18[...] = a
    o19[...] = b
    o20[...] = c
    o21[...] = d
    o22[...] = a * w_ref[18]
    o23[...] = b * w_ref[19]


def kernel(loss, prev_loss, weights, param_0, param_1, param_2, param_3,
           param_4, param_5, param_6, param_7, param_8, param_9, param_10,
           param_11, param_12, param_13, param_14, param_15):
    del loss, prev_loss
    params = (param_0, param_1, param_2, param_3, param_4, param_5, param_6,
              param_7, param_8, param_9, param_10, param_11, param_12,
              param_13, param_14, param_15)
    blk = pl.BlockSpec((_BLK, _COLS), lambda i: (i, 0))
    outs = pl.pallas_call(
        _tc_body,
        grid=(_ROWS // _BLK,),
        in_specs=[pl.BlockSpec(memory_space=pltpu.SMEM)] + [blk] * 16,
        out_specs=[blk] * 6,
        out_shape=[jax.ShapeDtypeStruct((_ROWS, _COLS), jnp.float32)] * 6,
    )(weights, *params)
    return tuple(outs)


# speedup vs baseline: 1.3088x; 1.0032x over previous
import jax
import jax.numpy as jnp
from jax.experimental import pallas as pl
from jax.experimental.pallas import tpu as pltpu

_ROWS = 2048
_COLS = 2048
_BLK = 64


def _tc_body(w_ref, p0, p1, p2, p3, p4, p5, p6, p7, p8, p9, p10, p11,
             p12, p13, p14, p15, o18, o19, o20, o21, o22, o23):
    a = p0[...] * w_ref[2] + p4[...] * w_ref[6] + p8[...] * w_ref[10] + p12[...] * w_ref[14]
    b = p1[...] * w_ref[3] + p5[...] * w_ref[7] + p9[...] * w_ref[11] + p13[...] * w_ref[15]
    c = p2[...] * w_ref[4] + p6[...] * w_ref[8] + p10[...] * w_ref[12] + p14[...] * w_ref[16]
    d = p3[...] * w_ref[5] + p7[...] * w_ref[9] + p11[...] * w_ref[13] + p15[...] * w_ref[17]
    o18[...] = a
    o19[...] = b
    o20[...] = c
    o21[...] = d
    o22[...] = a * w_ref[18]
    o23[...] = b * w_ref[19]


def kernel(loss, prev_loss, weights, param_0, param_1, param_2, param_3,
           param_4, param_5, param_6, param_7, param_8, param_9, param_10,
           param_11, param_12, param_13, param_14, param_15):
    del loss, prev_loss
    params = (param_0, param_1, param_2, param_3, param_4, param_5, param_6,
              param_7, param_8, param_9, param_10, param_11, param_12,
              param_13, param_14, param_15)
    blk = pl.BlockSpec((_BLK, _COLS), lambda i: (i, 0))
    outs = pl.pallas_call(
        _tc_body,
        grid=(_ROWS // _BLK,),
        in_specs=[pl.BlockSpec(memory_space=pltpu.SMEM)] + [blk] * 16,
        out_specs=[blk] * 6,
        out_shape=[jax.ShapeDtypeStruct((_ROWS, _COLS), jnp.float32)] * 6,
    )(weights, *params)
    return tuple(outs)
